# baseline (speedup 1.0000x reference)
import functools

import jax
import jax.numpy as jnp
from jax import lax
from jax.experimental import pallas as pl
from jax.experimental.pallas import tpu as pltpu
from jax.experimental.pallas import tpu_sc as plsc

N = 10000
E = 320000
D = 128
H = 8
C = 16

NC = 2
NS = 16
NW = NC * NS
EB = 128
ES = 79
EPW = ES * EB
EMAIN = 9984
ETAIL = 16
EPAD = EPW - EMAIN - ETAIL
NPAD = 10240
RPT = NPAD // NS
NB = 4

_mesh = plsc.VectorSubcoreMesh(core_axis_name="c", subcore_axis_name="s")
_sc_params = pltpu.CompilerParams(use_tc_tiling_on_sc=False)


def _load_pad_indices(w, main_src, main_dst, tail_src, tail_dst,
                      si_flat, di_2d, sem):
    pltpu.async_copy(main_src.at[pl.ds(w * EMAIN, EMAIN)],
                     si_flat.at[pl.ds(0, EMAIN)], sem)
    pltpu.async_copy(tail_src.at[w], si_flat.at[pl.ds(EMAIN, ETAIL)], sem)
    pltpu.async_copy(main_dst.at[w], di_2d.at[pl.ds(0, ES - 1)], sem)
    pltpu.async_copy(tail_dst.at[w], di_2d.at[ES - 1, pl.ds(0, ETAIL)], sem)
    base = w * EPAD
    for k in range(EPAD // 16):
        lane = base + k * 16 + lax.broadcasted_iota(jnp.int32, (16,), 0)
        si_flat[pl.ds(EMAIN + ETAIL + k * 16, 16)] = lax.rem(lane, N)
        di_2d[ES - 1, pl.ds(ETAIL + k * 16, 16)] = (
            N + lax.rem(lane, NPAD - N))
    pltpu.make_async_copy(main_src.at[pl.ds(0, EMAIN)],
                          si_flat.at[pl.ds(0, EMAIN)], sem).wait()
    pltpu.make_async_copy(tail_src.at[0],
                          si_flat.at[pl.ds(EMAIN, ETAIL)], sem).wait()
    pltpu.make_async_copy(main_dst.at[0], di_2d.at[pl.ds(0, ES - 1)],
                          sem).wait()
    pltpu.make_async_copy(tail_dst.at[0], di_2d.at[ES - 1, pl.ds(0, ETAIL)],
                          sem).wait()


@functools.partial(
    pl.kernel,
    out_type=jax.ShapeDtypeStruct((NC * NPAD,), jnp.float32),
    mesh=_mesh,
    compiler_params=_sc_params,
    scratch_types=[
        pltpu.VMEM((EPW,), jnp.int32),
        pltpu.VMEM((ES, EB), jnp.int32),
        pltpu.VMEM((EB,), jnp.float32),
        pltpu.VMEM_SHARED((NPAD,), jnp.float32),
        pltpu.SemaphoreType.DMA,
        pltpu.SemaphoreType.DMA,
    ],
)
def _sc_degree(main0_hbm, main1_hbm, tail0_hbm, tail1_hbm, zeros_hbm,
               ones_hbm, out_hbm, si_v, di_v, ones_v, acc, lsem, sem):
    c = lax.axis_index("c")
    s = lax.axis_index("s")
    w = c * NS + s
    pltpu.sync_copy(ones_hbm, ones_v)
    pltpu.sync_copy(zeros_hbm.at[pl.ds(s * RPT, RPT)],
                    acc.at[pl.ds(s * RPT, RPT)])
    _load_pad_indices(w, main0_hbm, main1_hbm, tail0_hbm, tail1_hbm,
                      si_v, di_v, lsem)
    plsc.subcore_barrier()

    def fire(j, _):
        pltpu.async_copy(ones_v, acc.at[di_v.at[j]], sem, add=True)
        return 0

    lax.fori_loop(0, ES, fire, 0)

    def drain(j, _):
        pltpu.make_async_copy(ones_v, acc.at[di_v.at[0]], sem).wait()
        return 0

    lax.fori_loop(0, ES, drain, 0)
    plsc.subcore_barrier()
    pltpu.sync_copy(acc.at[pl.ds(s * RPT, RPT)],
                    out_hbm.at[pl.ds(c * NPAD + s * RPT, RPT)])


def _make_sc_aggregate(F):
    @functools.partial(
        pl.kernel,
        out_type=jax.ShapeDtypeStruct((NC, NPAD, F), jnp.float32),
        mesh=_mesh,
        compiler_params=_sc_params,
        scratch_types=[
            pltpu.VMEM((EPW,), jnp.int32),
            pltpu.VMEM((ES, EB), jnp.int32),
            pltpu.VMEM((NB, EB, F), jnp.float32),
            pltpu.VMEM_SHARED((NPAD, F), jnp.float32),
            pltpu.VMEM_SHARED((N, F), jnp.float32),
            pltpu.SemaphoreType.DMA,
            pltpu.SemaphoreType.DMA,
            pltpu.SemaphoreType.DMA,
        ],
    )
    def agg(tab_hbm, main0_hbm, main1_hbm, tail0_hbm, tail1_hbm, zeros_hbm,
            out_hbm, si_v, di_v, rows_v, acc, tab_sh, lsem, gsem, ssem):
        c = lax.axis_index("c")
        s = lax.axis_index("s")
        w = c * NS + s
        TPT = N // NS
        pltpu.sync_copy(zeros_hbm.at[pl.ds(s * RPT, RPT)],
                        acc.at[pl.ds(s * RPT, RPT)])
        pltpu.sync_copy(tab_hbm.at[pl.ds(s * TPT, TPT)],
                        tab_sh.at[pl.ds(s * TPT, TPT)])
        _load_pad_indices(w, main0_hbm, main1_hbm, tail0_hbm, tail1_hbm,
                          si_v, di_v, lsem)
        plsc.subcore_barrier()

        for p in range(NB - 1):
            pltpu.async_copy(tab_sh.at[si_v.at[pl.ds(p * EB, EB)]],
                             rows_v.at[p], gsem)

        def step(j, _):
            b = lax.rem(j, NB)
            pltpu.make_async_copy(tab_sh.at[si_v.at[pl.ds(j * EB, EB)]],
                                  rows_v.at[b], gsem).wait()
            pltpu.async_copy(rows_v.at[b], acc.at[di_v.at[j]], ssem, add=True)

            @pl.when(j >= 1)
            def _():
                bp = lax.rem(j - 1, NB)
                pltpu.make_async_copy(rows_v.at[bp], acc.at[di_v.at[j - 1]],
                                      ssem).wait()

            @pl.when(j + NB - 1 < ES)
            def _():
                bn = lax.rem(j + NB - 1, NB)
                pltpu.async_copy(
                    tab_sh.at[si_v.at[pl.ds((j + NB - 1) * EB, EB)]],
                    rows_v.at[bn], gsem)

            return 0

        lax.fori_loop(0, ES, step, 0)
        pltpu.make_async_copy(rows_v.at[(ES - 1) % NB],
                              acc.at[di_v.at[ES - 1]], ssem).wait()
        plsc.subcore_barrier()
        pltpu.sync_copy(acc.at[pl.ds(s * RPT, RPT)],
                        out_hbm.at[c, pl.ds(s * RPT, RPT)])

    return agg


_sc_agg8 = _make_sc_aggregate(H)
_sc_agg16 = _make_sc_aggregate(C)


def _tc_prep_body(x_ref, w1_ref, degp_ref, xs_ref, dis_ref):
    deg = (degp_ref[pl.ds(0, N)] + degp_ref[pl.ds(NPAD, N)] + 1.0)
    dis = lax.rsqrt(deg).reshape(N, 1)
    xw = jnp.dot(x_ref[...], w1_ref[...],
                 preferred_element_type=jnp.float32)
    xs_ref[...] = xw * dis
    dis_ref[...] = dis


_tc_prep = pl.pallas_call(
    _tc_prep_body,
    out_shape=(jax.ShapeDtypeStruct((N, H), jnp.float32),
               jax.ShapeDtypeStruct((N, 1), jnp.float32)),
)


def _tc_mid_body(aggp_ref, xs_ref, dis_ref, b1_ref, w2_ref, out_ref):
    agg = aggp_ref[0, :N, :] + aggp_ref[1, :N, :] + xs_ref[...]
    h = jnp.maximum(dis_ref[...] * agg + b1_ref[...], 0.0)
    hw = jnp.dot(h, w2_ref[...], preferred_element_type=jnp.float32)
    out_ref[...] = hw * dis_ref[...]


_tc_mid = pl.pallas_call(
    _tc_mid_body,
    out_shape=jax.ShapeDtypeStruct((N, C), jnp.float32),
)


def _tc_out_body(aggp_ref, xs2_ref, dis_ref, b2_ref, out_ref):
    o = dis_ref[...] * (aggp_ref[0, :N, :] + aggp_ref[1, :N, :]
                        + xs2_ref[...]) + b2_ref[...]
    m = jnp.max(o, axis=1, keepdims=True)
    e = jnp.exp(o - m)
    lse = jnp.log(jnp.sum(e, axis=1, keepdims=True)) + m
    out_ref[...] = o - lse


_tc_out = pl.pallas_call(
    _tc_out_body,
    out_shape=jax.ShapeDtypeStruct((N, C), jnp.float32),
)


def kernel(x, edge_index, W1, b1, W2, b2):
    src = edge_index[0]
    dst = edge_index[1]
    main_src = src[:NW * EMAIN]
    main_dst = dst[:NW * EMAIN].reshape(NW, EMAIN // EB, EB)
    tail_src = src[NW * EMAIN:].reshape(NW, ETAIL)
    tail_dst = dst[NW * EMAIN:].reshape(NW, ETAIL)
    zeros8 = jnp.zeros((NPAD, H), jnp.float32)
    zeros16 = jnp.zeros((NPAD, C), jnp.float32)
    zeros1 = jnp.zeros((NPAD,), jnp.float32)
    ones = jnp.ones((EB,), jnp.float32)

    degp = _sc_degree(main_src, main_dst, tail_src, tail_dst, zeros1, ones)
    xs1, dis = _tc_prep(x, W1, degp)
    agg1 = _sc_agg8(xs1, main_src, main_dst, tail_src, tail_dst, zeros8)
    xs2 = _tc_mid(agg1, xs1, dis, b1.reshape(1, H), W2)
    agg2 = _sc_agg16(xs2, main_src, main_dst, tail_src, tail_dst, zeros16)
    return _tc_out(agg2, xs2, dis, b2.reshape(1, C))


# speedup vs baseline: 1.1357x; 1.1357x over previous
import functools

import jax
import jax.numpy as jnp
from jax import lax
from jax.experimental import pallas as pl
from jax.experimental.pallas import tpu as pltpu
from jax.experimental.pallas import tpu_sc as plsc

N = 10000
E = 320000
D = 128
H = 8
C = 16

NC = 2
NS = 16
NW = NC * NS
EB = 128
ES = 79
EPW = ES * EB
EMAIN = 9984
ETAIL = 16
EPAD = EPW - EMAIN - ETAIL
NPAD = 10240
RPT = NPAD // NS
NB = 4

_mesh = plsc.VectorSubcoreMesh(core_axis_name="c", subcore_axis_name="s")
_sc_params = pltpu.CompilerParams(use_tc_tiling_on_sc=False)


def _load_pad_indices(w, src_full, dst_full, si_flat, di_flat, sem):
    pltpu.async_copy(src_full.at[pl.ds(w * EMAIN, EMAIN)],
                     si_flat.at[pl.ds(0, EMAIN)], sem)
    pltpu.async_copy(src_full.at[pl.ds(NW * EMAIN + w * ETAIL, ETAIL)],
                     si_flat.at[pl.ds(EMAIN, ETAIL)], sem)
    pltpu.async_copy(dst_full.at[pl.ds(w * EMAIN, EMAIN)],
                     di_flat.at[pl.ds(0, EMAIN)], sem)
    pltpu.async_copy(dst_full.at[pl.ds(NW * EMAIN + w * ETAIL, ETAIL)],
                     di_flat.at[pl.ds(EMAIN, ETAIL)], sem)
    base = w * EPAD
    for k in range(EPAD // 16):
        lane = base + k * 16 + lax.broadcasted_iota(jnp.int32, (16,), 0)
        si_flat[pl.ds(EMAIN + ETAIL + k * 16, 16)] = lax.rem(lane, N)
        di_flat[pl.ds(EMAIN + ETAIL + k * 16, 16)] = (
            N + lax.rem(lane, NPAD - N))
    for _ in range(2):
        pltpu.make_async_copy(src_full.at[pl.ds(0, EMAIN)],
                              si_flat.at[pl.ds(0, EMAIN)], sem).wait()
        pltpu.make_async_copy(src_full.at[pl.ds(0, ETAIL)],
                              si_flat.at[pl.ds(EMAIN, ETAIL)], sem).wait()


@functools.partial(
    pl.kernel,
    out_type=jax.ShapeDtypeStruct((NC * NPAD,), jnp.float32),
    mesh=_mesh,
    compiler_params=_sc_params,
    scratch_types=[
        pltpu.VMEM((EPW,), jnp.int32),
        pltpu.VMEM((EPW,), jnp.int32),
        pltpu.VMEM((EB,), jnp.float32),
        pltpu.VMEM_SHARED((NPAD,), jnp.float32),
        pltpu.SemaphoreType.DMA,
        pltpu.SemaphoreType.DMA,
    ],
)
def _sc_degree(src_hbm, dst_hbm, zeros_hbm,
               ones_hbm, out_hbm, si_v, di_v, ones_v, acc, lsem, sem):
    c = lax.axis_index("c")
    s = lax.axis_index("s")
    w = c * NS + s
    pltpu.sync_copy(ones_hbm, ones_v)
    pltpu.sync_copy(zeros_hbm.at[pl.ds(s * RPT, RPT)],
                    acc.at[pl.ds(s * RPT, RPT)])
    _load_pad_indices(w, src_hbm, dst_hbm, si_v, di_v, lsem)
    plsc.subcore_barrier()

    def fire(j, _):
        pltpu.async_copy(ones_v, acc.at[di_v.at[pl.ds(j * EB, EB)]],
                         sem, add=True)
        return 0

    lax.fori_loop(0, ES, fire, 0)

    def drain(j, _):
        pltpu.make_async_copy(ones_v, acc.at[di_v.at[pl.ds(0, EB)]],
                              sem).wait()
        return 0

    lax.fori_loop(0, ES, drain, 0)
    plsc.subcore_barrier()
    pltpu.sync_copy(acc.at[pl.ds(s * RPT, RPT)],
                    out_hbm.at[pl.ds(c * NPAD + s * RPT, RPT)])


def _make_sc_aggregate(F):
    @functools.partial(
        pl.kernel,
        out_type=jax.ShapeDtypeStruct((NC, NPAD, F), jnp.float32),
        mesh=_mesh,
        compiler_params=_sc_params,
        scratch_types=[
            pltpu.VMEM((EPW,), jnp.int32),
            pltpu.VMEM((EPW,), jnp.int32),
            pltpu.VMEM((NB, EB, F), jnp.float32),
            pltpu.VMEM_SHARED((NPAD, F), jnp.float32),
            pltpu.VMEM_SHARED((N, F), jnp.float32),
            pltpu.SemaphoreType.DMA,
            pltpu.SemaphoreType.DMA,
            pltpu.SemaphoreType.DMA,
        ],
    )
    def agg(tab_hbm, src_hbm, dst_hbm, zeros_hbm,
            out_hbm, si_v, di_v, rows_v, acc, tab_sh, lsem, gsem, ssem):
        c = lax.axis_index("c")
        s = lax.axis_index("s")
        w = c * NS + s
        TPT = N // NS
        pltpu.sync_copy(zeros_hbm.at[pl.ds(s * RPT, RPT)],
                        acc.at[pl.ds(s * RPT, RPT)])
        pltpu.sync_copy(tab_hbm.at[pl.ds(s * TPT, TPT)],
                        tab_sh.at[pl.ds(s * TPT, TPT)])
        _load_pad_indices(w, src_hbm, dst_hbm, si_v, di_v, lsem)
        plsc.subcore_barrier()

        for p in range(NB - 1):
            pltpu.async_copy(tab_sh.at[si_v.at[pl.ds(p * EB, EB)]],
                             rows_v.at[p], gsem)

        def step(j, _):
            b = lax.rem(j, NB)
            pltpu.make_async_copy(tab_sh.at[si_v.at[pl.ds(j * EB, EB)]],
                                  rows_v.at[b], gsem).wait()
            pltpu.async_copy(rows_v.at[b], acc.at[di_v.at[pl.ds(j * EB, EB)]],
                             ssem, add=True)

            @pl.when(j >= 1)
            def _():
                bp = lax.rem(j - 1, NB)
                pltpu.make_async_copy(
                    rows_v.at[bp], acc.at[di_v.at[pl.ds((j - 1) * EB, EB)]],
                    ssem).wait()

            @pl.when(j + NB - 1 < ES)
            def _():
                bn = lax.rem(j + NB - 1, NB)
                pltpu.async_copy(
                    tab_sh.at[si_v.at[pl.ds((j + NB - 1) * EB, EB)]],
                    rows_v.at[bn], gsem)

            return 0

        lax.fori_loop(0, ES, step, 0)
        pltpu.make_async_copy(rows_v.at[(ES - 1) % NB],
                              acc.at[di_v.at[pl.ds((ES - 1) * EB, EB)]],
                              ssem).wait()
        plsc.subcore_barrier()
        pltpu.sync_copy(acc.at[pl.ds(s * RPT, RPT)],
                        out_hbm.at[c, pl.ds(s * RPT, RPT)])

    return agg


_sc_agg8 = _make_sc_aggregate(H)
_sc_agg16 = _make_sc_aggregate(C)


def _tc_prep_body(x_ref, w1_ref, degp_ref, xs_ref, dis_ref):
    deg = (degp_ref[pl.ds(0, N)] + degp_ref[pl.ds(NPAD, N)] + 1.0)
    dis = lax.rsqrt(deg).reshape(N, 1)
    xw = jnp.dot(x_ref[...], w1_ref[...],
                 preferred_element_type=jnp.float32)
    xs_ref[...] = xw * dis
    dis_ref[...] = dis


_tc_prep = pl.pallas_call(
    _tc_prep_body,
    out_shape=(jax.ShapeDtypeStruct((N, H), jnp.float32),
               jax.ShapeDtypeStruct((N, 1), jnp.float32)),
)


def _tc_mid_body(aggp_ref, xs_ref, dis_ref, b1_ref, w2_ref, out_ref):
    agg = aggp_ref[0, :N, :] + aggp_ref[1, :N, :] + xs_ref[...]
    h = jnp.maximum(dis_ref[...] * agg + b1_ref[...], 0.0)
    hw = jnp.dot(h, w2_ref[...], preferred_element_type=jnp.float32)
    out_ref[...] = hw * dis_ref[...]


_tc_mid = pl.pallas_call(
    _tc_mid_body,
    out_shape=jax.ShapeDtypeStruct((N, C), jnp.float32),
)


def _tc_out_body(aggp_ref, xs2_ref, dis_ref, b2_ref, out_ref):
    o = dis_ref[...] * (aggp_ref[0, :N, :] + aggp_ref[1, :N, :]
                        + xs2_ref[...]) + b2_ref[...]
    m = jnp.max(o, axis=1, keepdims=True)
    e = jnp.exp(o - m)
    lse = jnp.log(jnp.sum(e, axis=1, keepdims=True)) + m
    out_ref[...] = o - lse


_tc_out = pl.pallas_call(
    _tc_out_body,
    out_shape=jax.ShapeDtypeStruct((N, C), jnp.float32),
)


def kernel(x, edge_index, W1, b1, W2, b2):
    src = edge_index[0]
    dst = edge_index[1]
    zeros8 = jnp.zeros((NPAD, H), jnp.float32)
    zeros16 = jnp.zeros((NPAD, C), jnp.float32)
    zeros1 = jnp.zeros((NPAD,), jnp.float32)
    ones = jnp.ones((EB,), jnp.float32)

    degp = _sc_degree(src, dst, zeros1, ones)
    xs1, dis = _tc_prep(x, W1, degp)
    agg1 = _sc_agg8(xs1, src, dst, zeros8)
    xs2 = _tc_mid(agg1, xs1, dis, b1.reshape(1, H), W2)
    agg2 = _sc_agg16(xs2, src, dst, zeros16)
    return _tc_out(agg2, xs2, dis, b2.reshape(1, C))


# speedup vs baseline: 1.2274x; 1.0807x over previous
import functools

import jax
import jax.numpy as jnp
from jax import lax
from jax.experimental import pallas as pl
from jax.experimental.pallas import tpu as pltpu
from jax.experimental.pallas import tpu_sc as plsc

N = 10000
E = 320000
D = 128
H = 8
C = 16

NC = 2
NS = 16
NW = NC * NS
EB = 128
ES = 79
EPW = ES * EB
EMAIN = 9984
ETAIL = 16
EPAD = EPW - EMAIN - ETAIL
NPAD = 10240
RPT = NPAD // NS
NB = 4

_mesh = plsc.VectorSubcoreMesh(core_axis_name="c", subcore_axis_name="s")
_sc_params = pltpu.CompilerParams(use_tc_tiling_on_sc=False)


def _load_pad_indices(w, edge_hbm, si_flat, di_flat, sem):
    pltpu.async_copy(edge_hbm.at[0, pl.ds(w * EMAIN, EMAIN)],
                     si_flat.at[pl.ds(0, EMAIN)], sem)
    pltpu.async_copy(edge_hbm.at[0, pl.ds(NW * EMAIN + w * ETAIL, ETAIL)],
                     si_flat.at[pl.ds(EMAIN, ETAIL)], sem)
    pltpu.async_copy(edge_hbm.at[1, pl.ds(w * EMAIN, EMAIN)],
                     di_flat.at[pl.ds(0, EMAIN)], sem)
    pltpu.async_copy(edge_hbm.at[1, pl.ds(NW * EMAIN + w * ETAIL, ETAIL)],
                     di_flat.at[pl.ds(EMAIN, ETAIL)], sem)
    base = w * EPAD
    for k in range(EPAD // 16):
        lane = base + k * 16 + lax.broadcasted_iota(jnp.int32, (16,), 0)
        si_flat[pl.ds(EMAIN + ETAIL + k * 16, 16)] = lax.rem(lane, N)
        di_flat[pl.ds(EMAIN + ETAIL + k * 16, 16)] = (
            N + lax.rem(lane, NPAD - N))
    for _ in range(2):
        pltpu.make_async_copy(edge_hbm.at[0, pl.ds(0, EMAIN)],
                              si_flat.at[pl.ds(0, EMAIN)], sem).wait()
        pltpu.make_async_copy(edge_hbm.at[0, pl.ds(0, ETAIL)],
                              si_flat.at[pl.ds(EMAIN, ETAIL)], sem).wait()


@functools.partial(
    pl.kernel,
    out_type=jax.ShapeDtypeStruct((NC * NPAD,), jnp.float32),
    mesh=_mesh,
    compiler_params=_sc_params,
    scratch_types=[
        pltpu.VMEM((EPW,), jnp.int32),
        pltpu.VMEM((EPW,), jnp.int32),
        pltpu.VMEM((EB,), jnp.float32),
        pltpu.VMEM_SHARED((NPAD,), jnp.float32),
        pltpu.SemaphoreType.DMA,
        pltpu.SemaphoreType.DMA,
    ],
)
def _sc_degree(edge_hbm, zeros_hbm,
               ones_hbm, out_hbm, si_v, di_v, ones_v, acc, lsem, sem):
    c = lax.axis_index("c")
    s = lax.axis_index("s")
    w = c * NS + s
    pltpu.sync_copy(ones_hbm, ones_v)
    pltpu.sync_copy(zeros_hbm.at[pl.ds(s * RPT, RPT)],
                    acc.at[pl.ds(s * RPT, RPT)])
    _load_pad_indices(w, edge_hbm, si_v, di_v, lsem)
    plsc.subcore_barrier()

    def fire(j, _):
        pltpu.async_copy(ones_v, acc.at[di_v.at[pl.ds(j * EB, EB)]],
                         sem, add=True)
        return 0

    lax.fori_loop(0, ES, fire, 0)

    def drain(j, _):
        pltpu.make_async_copy(ones_v, acc.at[di_v.at[pl.ds(0, EB)]],
                              sem).wait()
        return 0

    lax.fori_loop(0, ES, drain, 0)
    plsc.subcore_barrier()
    pltpu.sync_copy(acc.at[pl.ds(s * RPT, RPT)],
                    out_hbm.at[pl.ds(c * NPAD + s * RPT, RPT)])


def _make_sc_aggregate(F):
    @functools.partial(
        pl.kernel,
        out_type=jax.ShapeDtypeStruct((NC, NPAD, F), jnp.float32),
        mesh=_mesh,
        compiler_params=_sc_params,
        scratch_types=[
            pltpu.VMEM((EPW,), jnp.int32),
            pltpu.VMEM((EPW,), jnp.int32),
            pltpu.VMEM((NB, EB, F), jnp.float32),
            pltpu.VMEM_SHARED((NPAD, F), jnp.float32),
            pltpu.VMEM_SHARED((N, F), jnp.float32),
            pltpu.SemaphoreType.DMA,
            pltpu.SemaphoreType.DMA,
            pltpu.SemaphoreType.DMA,
        ],
    )
    def agg(tab_hbm, edge_hbm, zeros_hbm,
            out_hbm, si_v, di_v, rows_v, acc, tab_sh, lsem, gsem, ssem):
        c = lax.axis_index("c")
        s = lax.axis_index("s")
        w = c * NS + s
        TPT = N // NS
        pltpu.sync_copy(zeros_hbm.at[pl.ds(s * RPT, RPT)],
                        acc.at[pl.ds(s * RPT, RPT)])
        pltpu.sync_copy(tab_hbm.at[pl.ds(s * TPT, TPT)],
                        tab_sh.at[pl.ds(s * TPT, TPT)])
        _load_pad_indices(w, edge_hbm, si_v, di_v, lsem)
        plsc.subcore_barrier()

        for p in range(NB - 1):
            pltpu.async_copy(tab_sh.at[si_v.at[pl.ds(p * EB, EB)]],
                             rows_v.at[p], gsem)

        def step(j, _):
            b = lax.rem(j, NB)
            pltpu.make_async_copy(tab_sh.at[si_v.at[pl.ds(j * EB, EB)]],
                                  rows_v.at[b], gsem).wait()
            pltpu.async_copy(rows_v.at[b], acc.at[di_v.at[pl.ds(j * EB, EB)]],
                             ssem, add=True)

            @pl.when(j >= 1)
            def _():
                bp = lax.rem(j - 1, NB)
                pltpu.make_async_copy(
                    rows_v.at[bp], acc.at[di_v.at[pl.ds((j - 1) * EB, EB)]],
                    ssem).wait()

            @pl.when(j + NB - 1 < ES)
            def _():
                bn = lax.rem(j + NB - 1, NB)
                pltpu.async_copy(
                    tab_sh.at[si_v.at[pl.ds((j + NB - 1) * EB, EB)]],
                    rows_v.at[bn], gsem)

            return 0

        lax.fori_loop(0, ES, step, 0)
        pltpu.make_async_copy(rows_v.at[(ES - 1) % NB],
                              acc.at[di_v.at[pl.ds((ES - 1) * EB, EB)]],
                              ssem).wait()
        plsc.subcore_barrier()
        pltpu.sync_copy(acc.at[pl.ds(s * RPT, RPT)],
                        out_hbm.at[c, pl.ds(s * RPT, RPT)])

    return agg


_sc_agg8 = _make_sc_aggregate(H)
_sc_agg16 = _make_sc_aggregate(C)


def _tc_prep_body(x_ref, w1_ref, degp_ref, xs_ref, dis_ref):
    deg = (degp_ref[pl.ds(0, N)] + degp_ref[pl.ds(NPAD, N)] + 1.0)
    dis = lax.rsqrt(deg).reshape(N, 1)
    xw = jnp.dot(x_ref[...], w1_ref[...],
                 preferred_element_type=jnp.float32)
    xs_ref[...] = xw * dis
    dis_ref[...] = dis


_tc_prep = pl.pallas_call(
    _tc_prep_body,
    out_shape=(jax.ShapeDtypeStruct((N, H), jnp.float32),
               jax.ShapeDtypeStruct((N, 1), jnp.float32)),
)


def _tc_mid_body(aggp_ref, xs_ref, dis_ref, b1_ref, w2_ref, out_ref):
    agg = aggp_ref[0, :N, :] + aggp_ref[1, :N, :] + xs_ref[...]
    h = jnp.maximum(dis_ref[...] * agg + b1_ref[...], 0.0)
    hw = jnp.dot(h, w2_ref[...], preferred_element_type=jnp.float32)
    out_ref[...] = hw * dis_ref[...]


_tc_mid = pl.pallas_call(
    _tc_mid_body,
    out_shape=jax.ShapeDtypeStruct((N, C), jnp.float32),
)


def _tc_out_body(aggp_ref, xs2_ref, dis_ref, b2_ref, out_ref):
    o = dis_ref[...] * (aggp_ref[0, :N, :] + aggp_ref[1, :N, :]
                        + xs2_ref[...]) + b2_ref[...]
    m = jnp.max(o, axis=1, keepdims=True)
    e = jnp.exp(o - m)
    lse = jnp.log(jnp.sum(e, axis=1, keepdims=True)) + m
    out_ref[...] = o - lse


_tc_out = pl.pallas_call(
    _tc_out_body,
    out_shape=jax.ShapeDtypeStruct((N, C), jnp.float32),
)


def kernel(x, edge_index, W1, b1, W2, b2):
    zeros8 = jnp.zeros((NPAD, H), jnp.float32)
    zeros16 = jnp.zeros((NPAD, C), jnp.float32)
    zeros1 = jnp.zeros((NPAD,), jnp.float32)
    ones = jnp.ones((EB,), jnp.float32)

    degp = _sc_degree(edge_index, zeros1, ones)
    xs1, dis = _tc_prep(x, W1, degp)
    agg1 = _sc_agg8(xs1, edge_index, zeros8)
    xs2 = _tc_mid(agg1, xs1, dis, b1.reshape(1, H), W2)
    agg2 = _sc_agg16(xs2, edge_index, zeros16)
    return _tc_out(agg2, xs2, dis, b2.reshape(1, C))


# speedup vs baseline: 1.2583x; 1.0252x over previous
import functools

import jax
import jax.numpy as jnp
from jax import lax
from jax.experimental import pallas as pl
from jax.experimental.pallas import tpu as pltpu
from jax.experimental.pallas import tpu_sc as plsc

N = 10000
E = 320000
D = 128
H = 8
C = 16

NC = 2
NS = 16
NW = NC * NS
EB = 128
ES = 79
EPW = ES * EB
EMAIN = 9984
ETAIL = 16
EPAD = EPW - EMAIN - ETAIL
NPAD = 10240
RPT = NPAD // NS
NB = 8
GA = 4

_mesh = plsc.VectorSubcoreMesh(core_axis_name="c", subcore_axis_name="s")
_sc_params = pltpu.CompilerParams(use_tc_tiling_on_sc=False)


def _load_pad_indices(w, edge_hbm, si_flat, di_flat, sem):
    pltpu.async_copy(edge_hbm.at[0, pl.ds(w * EMAIN, EMAIN)],
                     si_flat.at[pl.ds(0, EMAIN)], sem)
    pltpu.async_copy(edge_hbm.at[0, pl.ds(NW * EMAIN + w * ETAIL, ETAIL)],
                     si_flat.at[pl.ds(EMAIN, ETAIL)], sem)
    pltpu.async_copy(edge_hbm.at[1, pl.ds(w * EMAIN, EMAIN)],
                     di_flat.at[pl.ds(0, EMAIN)], sem)
    pltpu.async_copy(edge_hbm.at[1, pl.ds(NW * EMAIN + w * ETAIL, ETAIL)],
                     di_flat.at[pl.ds(EMAIN, ETAIL)], sem)
    base = w * EPAD
    for k in range(EPAD // 16):
        lane = base + k * 16 + lax.broadcasted_iota(jnp.int32, (16,), 0)
        si_flat[pl.ds(EMAIN + ETAIL + k * 16, 16)] = lax.rem(lane, N)
        di_flat[pl.ds(EMAIN + ETAIL + k * 16, 16)] = (
            N + lax.rem(lane, NPAD - N))
    for _ in range(2):
        pltpu.make_async_copy(edge_hbm.at[0, pl.ds(0, EMAIN)],
                              si_flat.at[pl.ds(0, EMAIN)], sem).wait()
        pltpu.make_async_copy(edge_hbm.at[0, pl.ds(0, ETAIL)],
                              si_flat.at[pl.ds(EMAIN, ETAIL)], sem).wait()


@functools.partial(
    pl.kernel,
    out_type=jax.ShapeDtypeStruct((NC * NPAD,), jnp.float32),
    mesh=_mesh,
    compiler_params=_sc_params,
    scratch_types=[
        pltpu.VMEM((EPW,), jnp.int32),
        pltpu.VMEM((EPW,), jnp.int32),
        pltpu.VMEM((EB,), jnp.float32),
        pltpu.VMEM((RPT,), jnp.float32),
        pltpu.VMEM_SHARED((NPAD,), jnp.float32),
        pltpu.SemaphoreType.DMA,
        pltpu.SemaphoreType.DMA,
    ],
)
def _sc_degree(edge_hbm, out_hbm, si_v, di_v, ones_v, zb_v, acc, lsem, sem):
    c = lax.axis_index("c")
    s = lax.axis_index("s")
    w = c * NS + s

    def fill(i, _):
        ones_v[pl.ds(i * 16, 16)] = jnp.ones((16,), jnp.float32)
        zb_v[pl.ds(i * 16, 16)] = jnp.zeros((16,), jnp.float32)
        zb_v[pl.ds((EB // 16 + i) * 16, 16)] = jnp.zeros((16,), jnp.float32)
        return 0

    lax.fori_loop(0, EB // 16, fill, 0)

    def zfill(i, _):
        zb_v[pl.ds(i * 16, 16)] = jnp.zeros((16,), jnp.float32)
        return 0

    lax.fori_loop(EB // 16, RPT // 16, zfill, 0)
    pltpu.sync_copy(zb_v, acc.at[pl.ds(s * RPT, RPT)])
    _load_pad_indices(w, edge_hbm, si_v, di_v, lsem)
    plsc.subcore_barrier()

    def fire(j, _):
        pltpu.async_copy(ones_v, acc.at[di_v.at[pl.ds(j * EB, EB)]],
                         sem, add=True)
        return 0

    lax.fori_loop(0, ES, fire, 0)

    def drain(j, _):
        pltpu.make_async_copy(ones_v, acc.at[di_v.at[pl.ds(0, EB)]],
                              sem).wait()
        return 0

    lax.fori_loop(0, ES, drain, 0)
    plsc.subcore_barrier()
    pltpu.sync_copy(acc.at[pl.ds(s * RPT, RPT)],
                    out_hbm.at[pl.ds(c * NPAD + s * RPT, RPT)])


def _make_sc_aggregate(F):
    @functools.partial(
        pl.kernel,
        out_type=jax.ShapeDtypeStruct((NC, NPAD, F), jnp.float32),
        mesh=_mesh,
        compiler_params=_sc_params,
        scratch_types=[
            pltpu.VMEM((EPW,), jnp.int32),
            pltpu.VMEM((EPW,), jnp.int32),
            pltpu.VMEM((NB, EB, F), jnp.float32),
            pltpu.VMEM_SHARED((NPAD, F), jnp.float32),
            pltpu.VMEM_SHARED((N, F), jnp.float32),
            pltpu.SemaphoreType.DMA,
            pltpu.SemaphoreType.DMA,
            pltpu.SemaphoreType.DMA,
        ],
    )
    def agg(tab_hbm, edge_hbm, zeros_hbm,
            out_hbm, si_v, di_v, rows_v, acc, tab_sh, lsem, gsem, ssem):
        c = lax.axis_index("c")
        s = lax.axis_index("s")
        w = c * NS + s
        TPT = N // NS
        pltpu.sync_copy(zeros_hbm.at[pl.ds(s * RPT, RPT)],
                        acc.at[pl.ds(s * RPT, RPT)])
        pltpu.sync_copy(tab_hbm.at[pl.ds(s * TPT, TPT)],
                        tab_sh.at[pl.ds(s * TPT, TPT)])
        _load_pad_indices(w, edge_hbm, si_v, di_v, lsem)
        plsc.subcore_barrier()

        for p in range(GA):
            pltpu.async_copy(tab_sh.at[si_v.at[pl.ds(p * EB, EB)]],
                             rows_v.at[p], gsem)

        def step(j, _):
            b = lax.rem(j, NB)
            pltpu.make_async_copy(tab_sh.at[si_v.at[pl.ds(j * EB, EB)]],
                                  rows_v.at[b], gsem).wait()
            pltpu.async_copy(rows_v.at[b], acc.at[di_v.at[pl.ds(j * EB, EB)]],
                             ssem, add=True)

            @pl.when(j >= GA)
            def _():
                bp = lax.rem(j - GA, NB)
                pltpu.make_async_copy(
                    rows_v.at[bp], acc.at[di_v.at[pl.ds((j - GA) * EB, EB)]],
                    ssem).wait()

            @pl.when(j + GA < ES)
            def _():
                bn = lax.rem(j + GA, NB)
                pltpu.async_copy(
                    tab_sh.at[si_v.at[pl.ds((j + GA) * EB, EB)]],
                    rows_v.at[bn], gsem)

            return 0

        lax.fori_loop(0, ES, step, 0)

        def sdrain(j, _):
            pltpu.make_async_copy(rows_v.at[0],
                                  acc.at[di_v.at[pl.ds(0, EB)]],
                                  ssem).wait()
            return 0

        lax.fori_loop(0, GA, sdrain, 0)
        plsc.subcore_barrier()
        pltpu.sync_copy(acc.at[pl.ds(s * RPT, RPT)],
                        out_hbm.at[c, pl.ds(s * RPT, RPT)])

    return agg


_sc_agg8 = _make_sc_aggregate(H)
_sc_agg16 = _make_sc_aggregate(C)


def _tc_prep_body(x_ref, w1_ref, degp_ref, xs_ref, dis_ref):
    deg = (degp_ref[pl.ds(0, N)] + degp_ref[pl.ds(NPAD, N)] + 1.0)
    dis = lax.rsqrt(deg).reshape(N, 1)
    xw = jnp.dot(x_ref[...], w1_ref[...],
                 preferred_element_type=jnp.float32)
    xs_ref[...] = xw * dis
    dis_ref[...] = dis


_tc_prep = pl.pallas_call(
    _tc_prep_body,
    out_shape=(jax.ShapeDtypeStruct((N, H), jnp.float32),
               jax.ShapeDtypeStruct((N, 1), jnp.float32)),
)


def _tc_mid_body(aggp_ref, xs_ref, dis_ref, b1_ref, w2_ref, out_ref):
    agg = aggp_ref[0, :N, :] + aggp_ref[1, :N, :] + xs_ref[...]
    h = jnp.maximum(dis_ref[...] * agg + b1_ref[...], 0.0)
    hw = jnp.dot(h, w2_ref[...], preferred_element_type=jnp.float32)
    out_ref[...] = hw * dis_ref[...]


_tc_mid = pl.pallas_call(
    _tc_mid_body,
    out_shape=jax.ShapeDtypeStruct((N, C), jnp.float32),
)


def _tc_out_body(aggp_ref, xs2_ref, dis_ref, b2_ref, out_ref):
    o = dis_ref[...] * (aggp_ref[0, :N, :] + aggp_ref[1, :N, :]
                        + xs2_ref[...]) + b2_ref[...]
    m = jnp.max(o, axis=1, keepdims=True)
    e = jnp.exp(o - m)
    lse = jnp.log(jnp.sum(e, axis=1, keepdims=True)) + m
    out_ref[...] = o - lse


_tc_out = pl.pallas_call(
    _tc_out_body,
    out_shape=jax.ShapeDtypeStruct((N, C), jnp.float32),
)


def kernel(x, edge_index, W1, b1, W2, b2):
    zeros8 = jnp.zeros((NPAD, H), jnp.float32)
    zeros16 = jnp.zeros((NPAD, C), jnp.float32)

    degp = _sc_degree(edge_index)
    xs1, dis = _tc_prep(x, W1, degp)
    agg1 = _sc_agg8(xs1, edge_index, zeros8)
    xs2 = _tc_mid(agg1, xs1, dis, b1.reshape(1, H), W2)
    agg2 = _sc_agg16(xs2, edge_index, zeros16)
    return _tc_out(agg2, xs2, dis, b2.reshape(1, C))


# speedup vs baseline: 1.4132x; 1.1231x over previous
import functools

import jax
import jax.numpy as jnp
from jax import lax
from jax.experimental import pallas as pl
from jax.experimental.pallas import tpu as pltpu
from jax.experimental.pallas import tpu_sc as plsc

N = 10000
E = 320000
D = 128
H = 8
C = 16

NC = 2
NS = 16
NW = NC * NS
EB = 128
ES = 79
EPW = ES * EB
EMAIN = 9984
ETAIL = 16
EPAD = EPW - EMAIN - ETAIL
NPAD = 10240
RPT = NPAD // NS
NB = 8
GA = 4

_mesh = plsc.VectorSubcoreMesh(core_axis_name="c", subcore_axis_name="s")
_sc_params = pltpu.CompilerParams(use_tc_tiling_on_sc=False)


def _load_pad_indices(w, edge_hbm, si_flat, di_flat, sem):
    pltpu.async_copy(edge_hbm.at[0, pl.ds(w * EMAIN, EMAIN)],
                     si_flat.at[pl.ds(0, EMAIN)], sem)
    pltpu.async_copy(edge_hbm.at[0, pl.ds(NW * EMAIN + w * ETAIL, ETAIL)],
                     si_flat.at[pl.ds(EMAIN, ETAIL)], sem)
    pltpu.async_copy(edge_hbm.at[1, pl.ds(w * EMAIN, EMAIN)],
                     di_flat.at[pl.ds(0, EMAIN)], sem)
    pltpu.async_copy(edge_hbm.at[1, pl.ds(NW * EMAIN + w * ETAIL, ETAIL)],
                     di_flat.at[pl.ds(EMAIN, ETAIL)], sem)
    base = w * EPAD
    for k in range(EPAD // 16):
        lane = base + k * 16 + lax.broadcasted_iota(jnp.int32, (16,), 0)
        si_flat[pl.ds(EMAIN + ETAIL + k * 16, 16)] = lax.rem(lane, N)
        di_flat[pl.ds(EMAIN + ETAIL + k * 16, 16)] = (
            N + lax.rem(lane, NPAD - N))
    for _ in range(2):
        pltpu.make_async_copy(edge_hbm.at[0, pl.ds(0, EMAIN)],
                              si_flat.at[pl.ds(0, EMAIN)], sem).wait()
        pltpu.make_async_copy(edge_hbm.at[0, pl.ds(0, ETAIL)],
                              si_flat.at[pl.ds(EMAIN, ETAIL)], sem).wait()


@functools.partial(
    pl.kernel,
    out_type=jax.ShapeDtypeStruct((NC * NPAD,), jnp.float32),
    mesh=_mesh,
    compiler_params=_sc_params,
    scratch_types=[
        pltpu.VMEM((EPW,), jnp.int32),
        pltpu.VMEM((EPW,), jnp.int32),
        pltpu.VMEM((EB,), jnp.float32),
        pltpu.VMEM((RPT,), jnp.float32),
        pltpu.VMEM_SHARED((NPAD,), jnp.float32),
        pltpu.SemaphoreType.DMA,
        pltpu.SemaphoreType.DMA,
    ],
)
def _sc_degree(edge_hbm, out_hbm, si_v, di_v, ones_v, zb_v, acc, lsem, sem):
    c = lax.axis_index("c")
    s = lax.axis_index("s")
    w = c * NS + s

    def fill(i, _):
        ones_v[pl.ds(i * 16, 16)] = jnp.ones((16,), jnp.float32)
        zb_v[pl.ds(i * 16, 16)] = jnp.zeros((16,), jnp.float32)
        zb_v[pl.ds((EB // 16 + i) * 16, 16)] = jnp.zeros((16,), jnp.float32)
        return 0

    lax.fori_loop(0, EB // 16, fill, 0)

    def zfill(i, _):
        zb_v[pl.ds(i * 16, 16)] = jnp.zeros((16,), jnp.float32)
        return 0

    lax.fori_loop(EB // 16, RPT // 16, zfill, 0)
    pltpu.sync_copy(zb_v, acc.at[pl.ds(s * RPT, RPT)])
    _load_pad_indices(w, edge_hbm, si_v, di_v, lsem)
    plsc.subcore_barrier()

    def fire(j, _):
        pltpu.async_copy(ones_v, acc.at[di_v.at[pl.ds(j * EB, EB)]],
                         sem, add=True)
        return 0

    lax.fori_loop(0, ES, fire, 0)

    def drain(j, _):
        pltpu.make_async_copy(ones_v, acc.at[di_v.at[pl.ds(0, EB)]],
                              sem).wait()
        return 0

    lax.fori_loop(0, ES, drain, 0)
    plsc.subcore_barrier()
    pltpu.sync_copy(acc.at[pl.ds(s * RPT, RPT)],
                    out_hbm.at[pl.ds(c * NPAD + s * RPT, RPT)])


def _make_sc_aggregate(F):
    @functools.partial(
        pl.kernel,
        out_type=jax.ShapeDtypeStruct((NC, NPAD, 128), jnp.float32),
        mesh=_mesh,
        compiler_params=_sc_params,
        scratch_types=[
            pltpu.VMEM((EPW,), jnp.int32),
            pltpu.VMEM((EPW,), jnp.int32),
            pltpu.VMEM((NB, EB, F), jnp.float32),
            pltpu.VMEM_SHARED((NPAD, F), jnp.float32),
            pltpu.VMEM_SHARED((N, F), jnp.float32),
            pltpu.SemaphoreType.DMA,
            pltpu.SemaphoreType.DMA,
            pltpu.SemaphoreType.DMA,
        ],
    )
    def agg(tab_hbm, edge_hbm, zeros_hbm,
            out_hbm, si_v, di_v, rows_v, acc, tab_sh, lsem, gsem, ssem):
        c = lax.axis_index("c")
        s = lax.axis_index("s")
        w = c * NS + s
        TPT = N // NS
        pltpu.sync_copy(zeros_hbm.at[pl.ds(s * RPT, RPT)],
                        acc.at[pl.ds(s * RPT, RPT)])
        pltpu.sync_copy(tab_hbm.at[pl.ds(s * TPT, TPT), pl.ds(0, F)],
                        tab_sh.at[pl.ds(s * TPT, TPT)])
        _load_pad_indices(w, edge_hbm, si_v, di_v, lsem)
        plsc.subcore_barrier()

        for p in range(GA):
            pltpu.async_copy(tab_sh.at[si_v.at[pl.ds(p * EB, EB)]],
                             rows_v.at[p], gsem)

        def step(j, _):
            b = lax.rem(j, NB)
            pltpu.make_async_copy(tab_sh.at[si_v.at[pl.ds(j * EB, EB)]],
                                  rows_v.at[b], gsem).wait()
            pltpu.async_copy(rows_v.at[b], acc.at[di_v.at[pl.ds(j * EB, EB)]],
                             ssem, add=True)

            @pl.when(j >= GA)
            def _():
                bp = lax.rem(j - GA, NB)
                pltpu.make_async_copy(
                    rows_v.at[bp], acc.at[di_v.at[pl.ds((j - GA) * EB, EB)]],
                    ssem).wait()

            @pl.when(j + GA < ES)
            def _():
                bn = lax.rem(j + GA, NB)
                pltpu.async_copy(
                    tab_sh.at[si_v.at[pl.ds((j + GA) * EB, EB)]],
                    rows_v.at[bn], gsem)

            return 0

        lax.fori_loop(0, ES, step, 0)

        def sdrain(j, _):
            pltpu.make_async_copy(rows_v.at[0],
                                  acc.at[di_v.at[pl.ds(0, EB)]],
                                  ssem).wait()
            return 0

        lax.fori_loop(0, GA, sdrain, 0)
        plsc.subcore_barrier()
        pltpu.sync_copy(acc.at[pl.ds(s * RPT, RPT)],
                        out_hbm.at[c, pl.ds(s * RPT, RPT), pl.ds(0, F)])

    return agg


_sc_agg8 = _make_sc_aggregate(H)
_sc_agg16 = _make_sc_aggregate(C)


def _tc_prep_body(x_ref, w1_ref, degp_ref, xs_ref, dis_ref):
    deg = (degp_ref[pl.ds(0, N)] + degp_ref[pl.ds(NPAD, N)] + 1.0)
    dis = lax.rsqrt(deg).reshape(N, 1)
    xw = jnp.dot(x_ref[...], w1_ref[...],
                 preferred_element_type=jnp.float32)
    xs_ref[...] = jnp.concatenate(
        [xw * dis, jnp.zeros((N, 128 - H), jnp.float32)], axis=1)
    dis_ref[...] = dis


_tc_prep = pl.pallas_call(
    _tc_prep_body,
    out_shape=(jax.ShapeDtypeStruct((N, 128), jnp.float32),
               jax.ShapeDtypeStruct((N, 1), jnp.float32)),
)


def _tc_mid_body(aggp_ref, xs_ref, dis_ref, b1_ref, w2_ref, out_ref):
    agg = (aggp_ref[0, :N, :H] + aggp_ref[1, :N, :H]
           + xs_ref[:, :H])
    h = jnp.maximum(dis_ref[...] * agg + b1_ref[...], 0.0)
    hw = jnp.dot(h, w2_ref[...], preferred_element_type=jnp.float32)
    out_ref[...] = jnp.concatenate(
        [hw * dis_ref[...], jnp.zeros((N, 128 - C), jnp.float32)], axis=1)


_tc_mid = pl.pallas_call(
    _tc_mid_body,
    out_shape=jax.ShapeDtypeStruct((N, 128), jnp.float32),
)


def _tc_out_body(aggp_ref, xs2_ref, dis_ref, b2_ref, out_ref):
    o = dis_ref[...] * (aggp_ref[0, :N, :C] + aggp_ref[1, :N, :C]
                        + xs2_ref[:, :C]) + b2_ref[...]
    m = jnp.max(o, axis=1, keepdims=True)
    e = jnp.exp(o - m)
    lse = jnp.log(jnp.sum(e, axis=1, keepdims=True)) + m
    out_ref[...] = o - lse


_tc_out = pl.pallas_call(
    _tc_out_body,
    out_shape=jax.ShapeDtypeStruct((N, C), jnp.float32),
)


def kernel(x, edge_index, W1, b1, W2, b2):
    zeros8 = jnp.zeros((NPAD, H), jnp.float32)
    zeros16 = jnp.zeros((NPAD, C), jnp.float32)

    degp = _sc_degree(edge_index)
    xs1, dis = _tc_prep(x, W1, degp)
    agg1 = _sc_agg8(xs1, edge_index, zeros8)
    xs2 = _tc_mid(agg1, xs1, dis, b1.reshape(1, H), W2)
    agg2 = _sc_agg16(xs2, edge_index, zeros16)
    return _tc_out(agg2, xs2, dis, b2.reshape(1, C))
